# baseline (speedup 1.0000x reference)
import jax
import jax.numpy as jnp
from jax.experimental import pallas as pl

_H = 8
_B = 8
_DH = 16


def _mm_bias(x, w, b):
    n, k = x.shape
    p = w.shape[1]
    bm = 1000
    assert n % bm == 0

    def body(x_ref, w_ref, b_ref, o_ref):
        o_ref[...] = (
            jnp.dot(x_ref[...], w_ref[...], preferred_element_type=jnp.float32)
            + b_ref[...]
        )

    return pl.pallas_call(
        body,
        grid=(n // bm,),
        in_specs=[
            pl.BlockSpec((bm, k), lambda i: (i, 0)),
            pl.BlockSpec((k, p), lambda i: (0, 0)),
            pl.BlockSpec((1, p), lambda i: (0, 0)),
        ],
        out_specs=pl.BlockSpec((bm, p), lambda i: (i, 0)),
        out_shape=jax.ShapeDtypeStruct((n, p), jnp.float32),
    )(x, w, b.reshape(1, p))


def kernel(x_author, x_field_of_study, x_institution, x_paper, src_author_affiliated_with_institution, dst_author_affiliated_with_institution, src_institution_to_author, dst_institution_to_author, src_author_writes_paper, dst_author_writes_paper, src_paper_to_author, dst_paper_to_author, src_paper_cites_paper, dst_paper_cites_paper, src_paper_has_topic_field_of_study, dst_paper_has_topic_field_of_study, src_field_of_study_to_paper, dst_field_of_study_to_paper, bases_weight, relW_author_affiliated_with_institution, relb_author_affiliated_with_institution, relW_institution_to_author, relb_institution_to_author, relW_author_writes_paper, relb_author_writes_paper, relW_paper_to_author, relb_paper_to_author, relW_paper_cites_paper, relb_paper_cites_paper, relW_paper_has_topic_field_of_study, relb_paper_has_topic_field_of_study, relW_field_of_study_to_paper, relb_field_of_study_to_paper, rootW_author, rootb_author, rootW_field_of_study, rootb_field_of_study, rootW_institution, rootb_institution, rootW_paper, rootb_paper):
    inp = dict(locals())
    node_types = ['author', 'field_of_study', 'institution', 'paper']
    edge_types = [
        ('author', 'affiliated_with', 'institution'),
        ('institution', 'to', 'author'),
        ('author', 'writes', 'paper'),
        ('paper', 'to', 'author'),
        ('paper', 'cites', 'paper'),
        ('paper', 'has_topic', 'field_of_study'),
        ('field_of_study', 'to', 'paper'),
    ]
    zero128 = jnp.zeros((bases_weight.shape[1],), dtype=jnp.float32)

    bases = {nt: _mm_bias(inp['x_' + nt], bases_weight, zero128)
             for nt in node_types}
    root = {}
    for nt in node_types:
        w = _mm_bias(inp['x_' + nt], inp['rootW_' + nt], inp['rootb_' + nt])
        w = w.reshape(-1, _H, _B)
        root[nt] = jnp.matmul(w, bases[nt].reshape(-1, _B, _DH))

    for (s, r, d) in edge_types:
        tag = s + '_' + r + '_' + d
        src = inp['src_' + tag]
        dst = inp['dst_' + tag]
        nd = inp['x_' + d].shape[0]
        msgs = bases[s][src]
        cnt = jax.ops.segment_sum(
            jnp.ones((msgs.shape[0],), dtype=jnp.float32), dst, num_segments=nd)
        agg_mean = jax.ops.segment_sum(
            msgs, dst, num_segments=nd) / jnp.maximum(cnt, 1.0)[:, None]
        agg_max = jax.ops.segment_max(msgs, dst, num_segments=nd)
        agg_max = jnp.where(cnt[:, None] > 0, agg_max, 0.0)
        agg = jnp.stack([agg_mean, agg_max], axis=1).reshape(-1, 2 * _B, _DH)
        w = _mm_bias(inp['x_' + d], inp['relW_' + tag], inp['relb_' + tag])
        w = w.reshape(-1, _H, 2 * _B)
        root[d] = root[d] + jnp.matmul(w, agg)

    return tuple(root[nt].reshape(-1, _H * _DH) for nt in node_types)
